# baseline (speedup 1.0000x reference)
import jax
import jax.numpy as jnp
from jax import lax
from jax.experimental import pallas as pl
from jax.experimental.pallas import tpu as pltpu
from jax.experimental.pallas import tpu_sc as plsc

B, L = 16384, 200
DAY_VOCAB, TIME_VOCAB = 7, 96
D = 64
N = B * L
NW = 32
POS_PER_W = N // NW
C = 256
G = C // 128
NCHUNKS = POS_PER_W // C


def _sc_body(dt_hbm, cat_hbm, out_hbm, *s):
    iraw, idxf, rows = (s[0], s[1]), (s[2], s[3]), (s[4], s[5])
    isem, gsem, wsem = (s[6], s[7]), (s[8], s[9]), (s[10], s[11])
    cat_sp = s[12]
    nc = 2
    wid = lax.axis_index("s") * nc + lax.axis_index("c")
    base0 = wid * POS_PER_W
    iota = lax.broadcasted_iota(jnp.int32, (16,), 0)
    even = iota * 2
    ngroups = NCHUNKS // 2

    @pl.when(lax.axis_index("s") == 0)
    def _():
        pltpu.sync_copy(cat_hbm, cat_sp)

    plsc.subcore_barrier()

    def idx_copy(b, ci):
        return pltpu.make_async_copy(
            dt_hbm.at[pl.ds(2 * (base0 + ci * C), 2 * C)], iraw[b], isem[b]
        )

    def write_copy(b, ci):
        return pltpu.make_async_copy(
            rows[b], out_hbm.at[pl.ds(base0 + ci * C, C)], wsem[b]
        )

    idx_copy(0, 0).start()
    idx_copy(1, 1).start()

    def group(g, carry):
        for b in range(2):
            ci = 2 * g + b
            idx_copy(b, ci).wait()
            for k in range(C // 16):
                w = k * 32 + even
                d = plsc.load_gather(iraw[b], [w])
                t = plsc.load_gather(iraw[b], [w + 1])
                j, c = k // 8, (k % 8) * 16
                idxf[b][j, pl.ds(c, 16)] = d * TIME_VOCAB + t

            @pl.when(g < ngroups - 1)
            def _():
                idx_copy(b, ci + 2).start()

            @pl.when(g >= 1)
            def _():
                write_copy(b, ci).wait()

            cps = [
                pltpu.async_copy(
                    cat_sp.at[idxf[b].at[gg]],
                    rows[b].at[pl.ds(gg * 128, 128)],
                    gsem[b],
                )
                for gg in range(G)
            ]
            for cp in cps:
                cp.wait()
            write_copy(b, ci).start()
        return carry

    lax.fori_loop(0, ngroups, group, None)
    write_copy(0, NCHUNKS - 2).wait()
    write_copy(1, NCHUNKS - 1).wait()


@jax.jit
def _daytime_sc(dt_flat, cat):
    mesh = plsc.VectorSubcoreMesh(core_axis_name="c", subcore_axis_name="s")
    return pl.kernel(
        _sc_body,
        out_type=jax.ShapeDtypeStruct((N, 2 * D), jnp.float32),
        mesh=mesh,
        compiler_params=pltpu.CompilerParams(
            needs_layout_passes=False, use_tc_tiling_on_sc=True
        ),
        scratch_types=(
            [pltpu.VMEM((2 * C,), jnp.int32)] * 2
            + [pltpu.VMEM((G, 128), jnp.int32)] * 2
            + [pltpu.VMEM((C, 2 * D), jnp.float32)] * 2
            + [pltpu.SemaphoreType.DMA] * 6
            + [pltpu.VMEM_SHARED((DAY_VOCAB * TIME_VOCAB, 2 * D), jnp.float32)]
        ),
    )(dt_flat, cat)


def kernel(daytime, emb_day, emb_time):
    cat = jnp.concatenate(
        (
            jnp.broadcast_to(emb_day[:, None, :], (DAY_VOCAB, TIME_VOCAB, D)),
            jnp.broadcast_to(emb_time[None, :, :], (DAY_VOCAB, TIME_VOCAB, D)),
        ),
        axis=-1,
    ).reshape(DAY_VOCAB * TIME_VOCAB, 2 * D)
    dt_flat = daytime.reshape(2 * N)
    out = _daytime_sc(dt_flat, cat)
    return out.reshape(B, L, 2 * D)


# speedup vs baseline: 4.4970x; 4.4970x over previous
import jax
import jax.numpy as jnp
from jax import lax
from jax.experimental import pallas as pl
from jax.experimental.pallas import tpu as pltpu
from jax.experimental.pallas import tpu_sc as plsc

B, L = 16384, 200
DAY_VOCAB, TIME_VOCAB = 7, 96
D = 64
N = B * L
NW = 32
GROUPS = B // 128
GPW = GROUPS // NW
BPG = 128
LPAD = 208


def _sc_body(dt_hbm, cat_hbm, out_hbm, *s):
    ibuf = s[0]
    idxf, rows = (s[1], s[2]), (s[3], s[4])
    gsem, wsem = (s[5], s[6]), (s[7], s[8])
    cat_sp = s[9]
    nc = 2
    wid = lax.axis_index("s") * nc + lax.axis_index("c")
    iota = lax.broadcasted_iota(jnp.int32, (16,), 0)

    @pl.when(lax.axis_index("s") == 0)
    def _():
        pltpu.sync_copy(cat_hbm, cat_sp)

    plsc.subcore_barrier()

    for g_i in range(GPW):
        gidx = wid * GPW + g_i
        pltpu.sync_copy(dt_hbm.at[:, gidx, :], ibuf.at[pl.ds(0, L)])

        def pair_body(pair, carry, g_i=g_i, gidx=gidx):
            for b in range(2):
                i = pair * 2 + b
                ivec = jnp.zeros((16,), jnp.int32) + i
                for k in range(13):
                    lvec = k * 16 + iota
                    d = plsc.load_gather(ibuf, [lvec, ivec])
                    t = plsc.load_gather(ibuf, [lvec, ivec + 128])
                    row, col = (0, k * 16) if k < 8 else (1, (k - 8) * 16)
                    idxf[b][row, pl.ds(col, 16)] = d * TIME_VOCAB + t

                bglob = gidx * BPG + i
                wcopy = pltpu.make_async_copy(
                    rows[b], out_hbm.at[pl.ds(bglob * L, L)], wsem[b]
                )
                if g_i == 0:
                    @pl.when(pair >= 1)
                    def _():
                        wcopy.wait()
                else:
                    wcopy.wait()
                c1 = pltpu.async_copy(
                    cat_sp.at[idxf[b].at[0]], rows[b].at[pl.ds(0, 128)], gsem[b]
                )
                c2 = pltpu.async_copy(
                    cat_sp.at[idxf[b].at[1, pl.ds(0, L - 128)]],
                    rows[b].at[pl.ds(128, L - 128)],
                    gsem[b],
                )
                c1.wait()
                c2.wait()
                wcopy.start()
            return carry

        lax.fori_loop(0, BPG // 2, pair_body, None)

    for b in range(2):
        last = (wid * GPW + GPW - 1) * BPG + (BPG - 2) + b
        pltpu.make_async_copy(
            rows[b], out_hbm.at[pl.ds(last * L, L)], wsem[b]
        ).wait()


@jax.jit
def _daytime_sc(dt3, cat):
    mesh = plsc.VectorSubcoreMesh(core_axis_name="c", subcore_axis_name="s")
    return pl.kernel(
        _sc_body,
        out_type=jax.ShapeDtypeStruct((N, 2 * D), jnp.float32),
        mesh=mesh,
        compiler_params=pltpu.CompilerParams(
            needs_layout_passes=False, use_tc_tiling_on_sc=True
        ),
        scratch_types=(
            [pltpu.VMEM((LPAD, 256), jnp.int32)]
            + [pltpu.VMEM((2, 128), jnp.int32)] * 2
            + [pltpu.VMEM((L, 2 * D), jnp.float32)] * 2
            + [pltpu.SemaphoreType.DMA] * 4
            + [pltpu.VMEM_SHARED((DAY_VOCAB * TIME_VOCAB, 2 * D), jnp.float32)]
        ),
    )(dt3, cat)


def kernel(daytime, emb_day, emb_time):
    cat = jnp.concatenate(
        (
            jnp.broadcast_to(emb_day[:, None, :], (DAY_VOCAB, TIME_VOCAB, D)),
            jnp.broadcast_to(emb_time[None, :, :], (DAY_VOCAB, TIME_VOCAB, D)),
        ),
        axis=-1,
    ).reshape(DAY_VOCAB * TIME_VOCAB, 2 * D)
    dt3 = (
        daytime.reshape(B // 128, 128, L, 2)
        .transpose(2, 0, 3, 1)
        .reshape(L, B // 128, 256)
    )
    out = _daytime_sc(dt3, cat)
    return out.reshape(B, L, 2 * D)


# speedup vs baseline: 4.7880x; 1.0647x over previous
import jax
import jax.numpy as jnp
from jax import lax
from jax.experimental import pallas as pl
from jax.experimental.pallas import tpu as pltpu
from jax.experimental.pallas import tpu_sc as plsc

B, L = 16384, 200
DAY_VOCAB, TIME_VOCAB = 7, 96
D = 64
N = B * L
NW = 32
GROUPS = B // 128
GPW = GROUPS // NW
BPG = 128
BPW = GPW * BPG
LPAD = 208
NBUF = 3


def _sc_body(dt_hbm, cat_hbm, out_hbm, *s):
    ibuf = s[0]
    idxf = s[1 : 1 + NBUF]
    rows = s[1 + NBUF : 1 + 2 * NBUF]
    gsem = s[1 + 2 * NBUF : 1 + 3 * NBUF]
    wsem = s[1 + 3 * NBUF : 1 + 4 * NBUF]
    cat_sp = s[1 + 4 * NBUF]
    nc = 2
    wid = lax.axis_index("s") * nc + lax.axis_index("c")
    base_b = wid * BPW
    iota = lax.broadcasted_iota(jnp.int32, (16,), 0)

    @pl.when(lax.axis_index("s") == 0)
    def _():
        pltpu.sync_copy(cat_hbm, cat_sp)

    plsc.subcore_barrier()

    def load_group(nb):
        gidx = wid * GPW + nb // BPG
        half = (nb // 64) % 2
        pltpu.sync_copy(
            dt_hbm.at[:, gidx, pl.ds(half * 64, 64)],
            ibuf.at[pl.ds(0, L), pl.ds(0, 64)],
        )
        pltpu.sync_copy(
            dt_hbm.at[:, gidx, pl.ds(128 + half * 64, 64)],
            ibuf.at[pl.ds(0, L), pl.ds(64, 64)],
        )

    def compute_idxf(nb, b):
        ivec = jnp.zeros((16,), jnp.int32) + (nb % 64)
        for k in range(13):
            lvec = k * 16 + iota
            d = plsc.load_gather(ibuf, [lvec, ivec])
            t = plsc.load_gather(ibuf, [lvec, ivec + 64])
            row, col = (0, k * 16) if k < 8 else (1, (k - 8) * 16)
            idxf[b][row, pl.ds(col, 16)] = d * TIME_VOCAB + t

    def gather_copies(b):
        return (
            pltpu.make_async_copy(
                cat_sp.at[idxf[b].at[0]], rows[b].at[pl.ds(0, 128)], gsem[b]
            ),
            pltpu.make_async_copy(
                cat_sp.at[idxf[b].at[1, pl.ds(0, L - 128)]],
                rows[b].at[pl.ds(128, L - 128)],
                gsem[b],
            ),
        )

    def wcopy(b, i):
        return pltpu.make_async_copy(
            rows[b], out_hbm.at[pl.ds((base_b + i) * L, L)], wsem[b]
        )

    load_group(0)
    for j in range(NBUF):
        compute_idxf(j, j)
        for c in gather_copies(j):
            c.start()

    def step(p, carry):
        for b in range(NBUF):
            i = NBUF * p + b

            @pl.when(i < BPW)
            def _(i=i, b=b):
                for c in gather_copies(b):
                    c.wait()
                wcopy(b, i).start()
                nb = i + NBUF

                @pl.when(nb < BPW)
                def _(nb=nb, b=b):
                    @pl.when(nb % 64 == 0)
                    def _():
                        load_group(nb)

                    compute_idxf(nb, b)
                    wcopy(b, nb).wait()
                    for c in gather_copies(b):
                        c.start()

        return carry

    lax.fori_loop(0, (BPW + NBUF - 1) // NBUF, step, None)
    for b in range(NBUF):
        wcopy(b, 0).wait()


@jax.jit
def _daytime_sc(dt3, cat):
    mesh = plsc.VectorSubcoreMesh(core_axis_name="c", subcore_axis_name="s")
    return pl.kernel(
        _sc_body,
        out_type=jax.ShapeDtypeStruct((N, 2 * D), jnp.float32),
        mesh=mesh,
        compiler_params=pltpu.CompilerParams(
            needs_layout_passes=False, use_tc_tiling_on_sc=True
        ),
        scratch_types=(
            [pltpu.VMEM((LPAD, 128), jnp.int32)]
            + [pltpu.VMEM((2, 128), jnp.int32)] * NBUF
            + [pltpu.VMEM((L, 2 * D), jnp.float32)] * NBUF
            + [pltpu.SemaphoreType.DMA] * (2 * NBUF)
            + [pltpu.VMEM_SHARED((DAY_VOCAB * TIME_VOCAB, 2 * D), jnp.float32)]
        ),
    )(dt3, cat)


def kernel(daytime, emb_day, emb_time):
    cat = jnp.concatenate(
        (
            jnp.broadcast_to(emb_day[:, None, :], (DAY_VOCAB, TIME_VOCAB, D)),
            jnp.broadcast_to(emb_time[None, :, :], (DAY_VOCAB, TIME_VOCAB, D)),
        ),
        axis=-1,
    ).reshape(DAY_VOCAB * TIME_VOCAB, 2 * D)
    dt3 = (
        daytime.reshape(B // 128, 128, L, 2)
        .transpose(2, 0, 3, 1)
        .reshape(L, B // 128, 256)
    )
    out = _daytime_sc(dt3, cat)
    return out.reshape(B, L, 2 * D)


# speedup vs baseline: 5.6975x; 1.1899x over previous
import jax
import jax.numpy as jnp
from jax import lax
from jax.experimental import pallas as pl
from jax.experimental.pallas import tpu as pltpu
from jax.experimental.pallas import tpu_sc as plsc

B, L = 16384, 200
DAY_VOCAB, TIME_VOCAB = 7, 96
D = 64
N = B * L
NW = 32
GROUPS = B // 128
GPW = GROUPS // NW
ROWS_PER_W = N // NW
UNIT = 128
UPW = ROWS_PER_W // UNIT
UPG = 128 * L // UNIT
NBUF = 4


def _sc_body(dt_hbm, cat_hbm, out_hbm, *s):
    ibuf = s[0]
    idxf = s[1 : 1 + NBUF]
    rows = s[1 + NBUF : 1 + 2 * NBUF]
    gsem = s[1 + 2 * NBUF : 1 + 3 * NBUF]
    wsem = s[1 + 3 * NBUF : 1 + 4 * NBUF]
    cat_sp = s[1 + 4 * NBUF]
    nc = 2
    wid = lax.axis_index("s") * nc + lax.axis_index("c")
    row0_w = wid * ROWS_PER_W
    iota = lax.broadcasted_iota(jnp.int32, (16,), 0)

    @pl.when(lax.axis_index("s") == 0)
    def _():
        pltpu.sync_copy(cat_hbm, cat_sp)

    plsc.subcore_barrier()

    def load_group(nu):
        gidx = wid * GPW + nu // UPG
        pltpu.sync_copy(dt_hbm.at[:, gidx, :], ibuf)

    def compute_idxf(nu, b):
        r0 = row0_w + nu * UNIT
        for k in range(UNIT // 16):
            r = r0 + k * 16 + iota
            bat = r // L
            l = r - bat * L
            col = bat & 127
            d = plsc.load_gather(ibuf, [l, col])
            t = plsc.load_gather(ibuf, [l, col + 128])
            idxf[b][0, pl.ds(k * 16, 16)] = d * TIME_VOCAB + t

    def gcopy(b):
        return pltpu.make_async_copy(
            cat_sp.at[idxf[b].at[0]], rows[b], gsem[b]
        )

    def wcopy(b, u):
        return pltpu.make_async_copy(
            rows[b], out_hbm.at[pl.ds(row0_w + u * UNIT, UNIT)], wsem[b]
        )

    load_group(0)
    for j in range(NBUF):
        compute_idxf(j, j)
        gcopy(j).start()

    def step(p, carry):
        for b in range(NBUF):
            u = NBUF * p + b
            gcopy(b).wait()
            wcopy(b, u).start()
            nu = u + NBUF

            @pl.when(nu < UPW)
            def _(nu=nu, b=b):
                @pl.when(nu % UPG == 0)
                def _():
                    load_group(nu)

                compute_idxf(nu, b)
                wcopy(b, nu).wait()
                gcopy(b).start()

        return carry

    lax.fori_loop(0, UPW // NBUF, step, None)
    for b in range(NBUF):
        wcopy(b, 0).wait()


@jax.jit
def _daytime_sc(dt3, cat):
    mesh = plsc.VectorSubcoreMesh(core_axis_name="c", subcore_axis_name="s")
    return pl.kernel(
        _sc_body,
        out_type=jax.ShapeDtypeStruct((N, 2 * D), jnp.float32),
        mesh=mesh,
        compiler_params=pltpu.CompilerParams(
            needs_layout_passes=False, use_tc_tiling_on_sc=True
        ),
        scratch_types=(
            [pltpu.VMEM((L, 256), jnp.int32)]
            + [pltpu.VMEM((1, 128), jnp.int32)] * NBUF
            + [pltpu.VMEM((UNIT, 2 * D), jnp.float32)] * NBUF
            + [pltpu.SemaphoreType.DMA] * (2 * NBUF)
            + [pltpu.VMEM_SHARED((DAY_VOCAB * TIME_VOCAB, 2 * D), jnp.float32)]
        ),
    )(dt3, cat)


def kernel(daytime, emb_day, emb_time):
    cat = jnp.concatenate(
        (
            jnp.broadcast_to(emb_day[:, None, :], (DAY_VOCAB, TIME_VOCAB, D)),
            jnp.broadcast_to(emb_time[None, :, :], (DAY_VOCAB, TIME_VOCAB, D)),
        ),
        axis=-1,
    ).reshape(DAY_VOCAB * TIME_VOCAB, 2 * D)
    dt3 = (
        daytime.reshape(B // 128, 128, L, 2)
        .transpose(2, 0, 3, 1)
        .reshape(L, B // 128, 256)
    )
    out = _daytime_sc(dt3, cat)
    return out.reshape(B, L, 2 * D)
